# baseline (device time: 69794 ns/iter reference)
import jax
import jax.numpy as jnp
from jax import lax
from jax.experimental import pallas as pl
from jax.experimental.pallas import tpu as pltpu

B, S, D = 2, 512, 2048
DC = 256
DC_SH = 128
H, DH, DR = 16, 128, 32
DP = 256
PAD = DP - DH - DR
BS = B * S
SCALE = (DH + DR) ** -0.5
NT = 4
TD = D // NT
HT = H // NT

_VMEM_LIMIT = 100 * 1024 * 1024


def _gather_q_body(x_ref, wdkv_ref, wuk_ref, wuv_ref, wq_ref, wqr_ref,
                   wkr_ref, wo_ref,
                   c_ref, wukf_ref, wuvf_ref, q_ref, qr_ref, kr_ref,
                   wob_ref, xb_ref, send_sems, recv_sems):
    my_x = lax.axis_index("x")
    my_y = lax.axis_index("y")
    nbr = (my_x, 1 - my_y)
    off = my_y * DC_SH
    j = pl.program_id(0)

    def _copies():
        return [
            pltpu.make_async_remote_copy(
                src_ref=c_ref.at[:, pl.ds(off, DC_SH)],
                dst_ref=c_ref.at[:, pl.ds(off, DC_SH)],
                send_sem=send_sems.at[0], recv_sem=recv_sems.at[0],
                device_id=nbr, device_id_type=pl.DeviceIdType.MESH),
            pltpu.make_async_remote_copy(
                src_ref=wukf_ref.at[pl.ds(off, DC_SH), :],
                dst_ref=wukf_ref.at[pl.ds(off, DC_SH), :],
                send_sem=send_sems.at[1], recv_sem=recv_sems.at[1],
                device_id=nbr, device_id_type=pl.DeviceIdType.MESH),
            pltpu.make_async_remote_copy(
                src_ref=wuvf_ref.at[pl.ds(off, DC_SH), :],
                dst_ref=wuvf_ref.at[pl.ds(off, DC_SH), :],
                send_sem=send_sems.at[2], recv_sem=recv_sems.at[2],
                device_id=nbr, device_id_type=pl.DeviceIdType.MESH),
        ]

    @pl.when(j == 0)
    def _():
        barrier = pltpu.get_barrier_semaphore()
        pl.semaphore_signal(barrier, inc=1, device_id=nbr,
                            device_id_type=pl.DeviceIdType.MESH)
        pl.semaphore_wait(barrier, 1)

        xb_ref[...] = x_ref[...].reshape(BS, D).astype(jnp.bfloat16)
        c_loc = jnp.dot(xb_ref[...], wdkv_ref[...].astype(jnp.bfloat16),
                        preferred_element_type=jnp.float32)
        c_ref[:, pl.ds(off, DC_SH)] = c_loc.astype(jnp.bfloat16)
        wukf_ref[pl.ds(off, DC_SH), :] = wuk_ref[...].astype(jnp.bfloat16)
        wuvf_ref[pl.ds(off, DC_SH), :] = wuv_ref[...].astype(jnp.bfloat16)
        for cp in _copies():
            cp.start()

        kr_ref[...] = jnp.dot(xb_ref[...], wkr_ref[...].astype(jnp.bfloat16),
                              preferred_element_type=jnp.float32
                              ).astype(jnp.bfloat16)
        qr_ref[...] = (jnp.dot(xb_ref[...],
                               wqr_ref[...].astype(jnp.bfloat16),
                               preferred_element_type=jnp.float32)
                       * SCALE).astype(jnp.bfloat16)

    q_ref[...] = (jnp.dot(xb_ref[...], wq_ref[...].astype(jnp.bfloat16),
                          preferred_element_type=jnp.float32)
                  * SCALE).astype(jnp.bfloat16)
    wob_ref[...] = wo_ref[...].astype(jnp.bfloat16)

    @pl.when(j == NT - 1)
    def _():
        for cp in _copies():
            cp.wait()


def _gather_q(x, wdkv, wuk, wuv, wq, wqr, wkr, wo):
    return pl.pallas_call(
        _gather_q_body,
        grid=(NT,),
        in_specs=[
            pl.BlockSpec((B, S, D), lambda j: (0, 0, 0)),
            pl.BlockSpec((D, DC_SH), lambda j: (0, 0)),
            pl.BlockSpec((DC_SH, D), lambda j: (0, 0)),
            pl.BlockSpec((DC_SH, D), lambda j: (0, 0)),
            pl.BlockSpec((D, TD), lambda j: (0, j)),
            pl.BlockSpec((D, H * DR), lambda j: (0, 0)),
            pl.BlockSpec((D, DR), lambda j: (0, 0)),
            pl.BlockSpec((D, TD), lambda j: (0, j)),
        ],
        out_specs=(
            pl.BlockSpec((BS, DC), lambda j: (0, 0)),
            pl.BlockSpec((DC, D), lambda j: (0, 0)),
            pl.BlockSpec((DC, D), lambda j: (0, 0)),
            pl.BlockSpec((BS, TD), lambda j: (0, j)),
            pl.BlockSpec((BS, H * DR), lambda j: (0, 0)),
            pl.BlockSpec((BS, DR), lambda j: (0, 0)),
            pl.BlockSpec((D, TD), lambda j: (0, j)),
        ),
        out_shape=(
            jax.ShapeDtypeStruct((BS, DC), jnp.bfloat16),
            jax.ShapeDtypeStruct((DC, D), jnp.bfloat16),
            jax.ShapeDtypeStruct((DC, D), jnp.bfloat16),
            jax.ShapeDtypeStruct((BS, D), jnp.bfloat16),
            jax.ShapeDtypeStruct((BS, H * DR), jnp.bfloat16),
            jax.ShapeDtypeStruct((BS, DR), jnp.bfloat16),
            jax.ShapeDtypeStruct((D, D), jnp.bfloat16),
        ),
        scratch_shapes=[
            pltpu.VMEM((BS, D), jnp.bfloat16),
            pltpu.SemaphoreType.DMA((3,)),
            pltpu.SemaphoreType.DMA((3,)),
        ],
        compiler_params=pltpu.CompilerParams(
            collective_id=0, vmem_limit_bytes=_VMEM_LIMIT),
    )(x, wdkv, wuk, wuv, wq, wqr, wkr, wo)


def _attn_body(c_ref, wuk_ref, wuv_ref, q_ref, qr_ref, kr_ref, wo_ref,
               out_ref, acc_ref):
    g = pl.program_id(1)
    c = c_ref[...]
    kk = jnp.dot(c, wuk_ref[...],
                 preferred_element_type=jnp.float32).astype(jnp.bfloat16)
    vv = jnp.dot(c, wuv_ref[...],
                 preferred_element_type=jnp.float32).astype(jnp.bfloat16)
    kr = kr_ref[...]
    zpad = jnp.zeros((S, PAD), jnp.bfloat16)
    dn = (((1,), (1,)), ((), ()))
    os = []
    for i in range(HT):
        qc = jnp.concatenate(
            [q_ref[:, i * DH:(i + 1) * DH],
             qr_ref[:, i * DR:(i + 1) * DR], zpad], axis=1)
        kc = jnp.concatenate(
            [kk[:, i * DH:(i + 1) * DH], kr, zpad], axis=1)
        s = lax.dot_general(qc, kc, dn, preferred_element_type=jnp.float32)
        p = jnp.exp(s.astype(jnp.bfloat16))
        rs = 1.0 / jnp.sum(p, axis=-1, keepdims=True, dtype=jnp.float32)
        o = jnp.dot(p, vv[:, i * DH:(i + 1) * DH],
                    preferred_element_type=jnp.float32)
        os.append((o * rs).astype(jnp.bfloat16))
    proj = jnp.dot(jnp.concatenate(os, axis=1), wo_ref[...],
                   preferred_element_type=jnp.float32)

    @pl.when(g == 0)
    def _():
        acc_ref[...] = proj

    @pl.when(g != 0)
    def _():
        acc_ref[...] = acc_ref[...] + proj

    @pl.when(g == NT - 1)
    def _():
        out_ref[...] = acc_ref[...].reshape(1, S, D)


def _attention(c, wuk_f, wuv_f, q, qr, kr, wo_b):
    return pl.pallas_call(
        _attn_body,
        grid=(B, NT),
        in_specs=[
            pl.BlockSpec((S, DC), lambda b, g: (b, 0)),
            pl.BlockSpec((DC, TD), lambda b, g: (0, g)),
            pl.BlockSpec((DC, TD), lambda b, g: (0, g)),
            pl.BlockSpec((S, HT * DH), lambda b, g: (b, g)),
            pl.BlockSpec((S, HT * DR), lambda b, g: (b, g)),
            pl.BlockSpec((S, DR), lambda b, g: (b, 0)),
            pl.BlockSpec((TD, D), lambda b, g: (g, 0)),
        ],
        out_specs=pl.BlockSpec((1, S, D), lambda b, g: (b, 0, 0)),
        out_shape=jax.ShapeDtypeStruct((B, S, D), jnp.float32),
        scratch_shapes=[
            pltpu.VMEM((S, D), jnp.float32),
        ],
        compiler_params=pltpu.CompilerParams(vmem_limit_bytes=_VMEM_LIMIT),
    )(c, wuk_f, wuv_f, q, qr, kr, wo_b)


def kernel(x, Wdkv, Wuk, Wuv, Wq, Wqr, Wkr, Wo):
    c, wuk_f, wuv_f, q, qr, kr, wo_b = _gather_q(
        x, Wdkv, Wuk, Wuv, Wq, Wqr, Wkr, Wo)
    return _attention(c, wuk_f, wuv_f, q, qr, kr, wo_b)


# device time: 64298 ns/iter; 1.0855x vs baseline; 1.0855x over previous
import jax
import jax.numpy as jnp
from jax import lax
from jax.experimental import pallas as pl
from jax.experimental.pallas import tpu as pltpu

B, S, D = 2, 512, 2048
DC = 256
DC_SH = 128
H, DH, DR = 16, 128, 32
DP = 256
PAD = DP - DH - DR
BS = B * S
SCALE = (DH + DR) ** -0.5
NT = 8
TD = D // NT

_VMEM_LIMIT = 100 * 1024 * 1024


def _body(x_ref, wdkv_ref, wuk_ref, wuv_ref, wq_ref, wqr_ref, wkr_ref,
          wo_ref, out_ref,
          xb_ref, c_ref, wukf_ref, wuvf_ref, q_ref, qr_ref, kr_ref,
          wob_ref, o_scratch, send_sems, recv_sems):
    my_x = lax.axis_index("x")
    my_y = lax.axis_index("y")
    nbr = (my_x, 1 - my_y)
    off = my_y * DC_SH
    j = pl.program_id(0)

    def _copies():
        return [
            pltpu.make_async_remote_copy(
                src_ref=c_ref.at[:, pl.ds(off, DC_SH)],
                dst_ref=c_ref.at[:, pl.ds(off, DC_SH)],
                send_sem=send_sems.at[0], recv_sem=recv_sems.at[0],
                device_id=nbr, device_id_type=pl.DeviceIdType.MESH),
            pltpu.make_async_remote_copy(
                src_ref=wukf_ref.at[pl.ds(off, DC_SH), :],
                dst_ref=wukf_ref.at[pl.ds(off, DC_SH), :],
                send_sem=send_sems.at[1], recv_sem=recv_sems.at[1],
                device_id=nbr, device_id_type=pl.DeviceIdType.MESH),
            pltpu.make_async_remote_copy(
                src_ref=wuvf_ref.at[pl.ds(off, DC_SH), :],
                dst_ref=wuvf_ref.at[pl.ds(off, DC_SH), :],
                send_sem=send_sems.at[2], recv_sem=recv_sems.at[2],
                device_id=nbr, device_id_type=pl.DeviceIdType.MESH),
        ]

    @pl.when(j == 0)
    def _():
        barrier = pltpu.get_barrier_semaphore()
        pl.semaphore_signal(barrier, inc=1, device_id=nbr,
                            device_id_type=pl.DeviceIdType.MESH)
        pl.semaphore_wait(barrier, 1)

        xb_ref[...] = x_ref[...].reshape(BS, D).astype(jnp.bfloat16)
        c_loc = jnp.dot(xb_ref[...], wdkv_ref[...].astype(jnp.bfloat16),
                        preferred_element_type=jnp.float32)
        c_ref[:, pl.ds(off, DC_SH)] = c_loc.astype(jnp.bfloat16)
        wukf_ref[pl.ds(off, DC_SH), :] = wuk_ref[...].astype(jnp.bfloat16)
        wuvf_ref[pl.ds(off, DC_SH), :] = wuv_ref[...].astype(jnp.bfloat16)
        for cp in _copies():
            cp.start()

        kr_ref[...] = jnp.dot(xb_ref[...], wkr_ref[...].astype(jnp.bfloat16),
                              preferred_element_type=jnp.float32
                              ).astype(jnp.bfloat16)
        qr_ref[...] = (jnp.dot(xb_ref[...],
                               wqr_ref[...].astype(jnp.bfloat16),
                               preferred_element_type=jnp.float32)
                       * SCALE).astype(jnp.bfloat16)

    @pl.when(j < NT)
    def _():
        jc = jnp.minimum(j, NT - 1)
        q_ref[:, pl.ds(jc * TD, TD)] = (
            jnp.dot(xb_ref[...], wq_ref[...].astype(jnp.bfloat16),
                    preferred_element_type=jnp.float32) * SCALE
        ).astype(jnp.bfloat16)
        wob_ref[:, pl.ds(jc * TD, TD)] = wo_ref[...].astype(jnp.bfloat16)

    @pl.when(j == NT - 1)
    def _():
        for cp in _copies():
            cp.wait()

    @pl.when(j >= NT)
    def _():
        b = jnp.maximum(j - NT, 0)
        rows = pl.ds(b * S, S)
        c = c_ref[rows, :]
        kk = jnp.dot(c, wukf_ref[...],
                     preferred_element_type=jnp.float32).astype(jnp.bfloat16)
        vv = jnp.dot(c, wuvf_ref[...],
                     preferred_element_type=jnp.float32).astype(jnp.bfloat16)
        kr = kr_ref[rows, :]
        qq = q_ref[rows, :]
        qr = qr_ref[rows, :]
        zpad = jnp.zeros((S, PAD), jnp.bfloat16)
        dn = (((1,), (1,)), ((), ()))
        for h in range(H):
            qc = jnp.concatenate(
                [qq[:, h * DH:(h + 1) * DH],
                 qr[:, h * DR:(h + 1) * DR], zpad], axis=1)
            kc = jnp.concatenate(
                [kk[:, h * DH:(h + 1) * DH], kr, zpad], axis=1)
            s = lax.dot_general(qc, kc, dn,
                                preferred_element_type=jnp.float32)
            p = jnp.exp(s.astype(jnp.bfloat16))
            rs = 1.0 / jnp.sum(p, axis=-1, keepdims=True,
                               dtype=jnp.float32)
            o = jnp.dot(p, vv[:, h * DH:(h + 1) * DH],
                        preferred_element_type=jnp.float32)
            o_scratch[:, h * DH:(h + 1) * DH] = (o * rs).astype(jnp.bfloat16)
        out_ref[...] = jnp.dot(o_scratch[...], wob_ref[...],
                               preferred_element_type=jnp.float32
                               ).reshape(1, S, D)


def kernel(x, Wdkv, Wuk, Wuv, Wq, Wqr, Wkr, Wo):
    return pl.pallas_call(
        _body,
        grid=(NT + B,),
        in_specs=[
            pl.BlockSpec((B, S, D), lambda j: (0, 0, 0)),
            pl.BlockSpec((D, DC_SH), lambda j: (0, 0)),
            pl.BlockSpec((DC_SH, D), lambda j: (0, 0)),
            pl.BlockSpec((DC_SH, D), lambda j: (0, 0)),
            pl.BlockSpec((D, TD),
                         lambda j: (0, jnp.minimum(j, NT - 1))),
            pl.BlockSpec((D, H * DR), lambda j: (0, 0)),
            pl.BlockSpec((D, DR), lambda j: (0, 0)),
            pl.BlockSpec((D, TD),
                         lambda j: (0, jnp.minimum(j, NT - 1))),
        ],
        out_specs=pl.BlockSpec(
            (1, S, D), lambda j: (jnp.maximum(j - NT, 0), 0, 0)),
        out_shape=jax.ShapeDtypeStruct((B, S, D), jnp.float32),
        scratch_shapes=[
            pltpu.VMEM((BS, D), jnp.bfloat16),
            pltpu.VMEM((BS, DC), jnp.bfloat16),
            pltpu.VMEM((DC, D), jnp.bfloat16),
            pltpu.VMEM((DC, D), jnp.bfloat16),
            pltpu.VMEM((BS, D), jnp.bfloat16),
            pltpu.VMEM((BS, H * DR), jnp.bfloat16),
            pltpu.VMEM((BS, DR), jnp.bfloat16),
            pltpu.VMEM((D, D), jnp.bfloat16),
            pltpu.VMEM((S, H * DH), jnp.bfloat16),
            pltpu.SemaphoreType.DMA((3,)),
            pltpu.SemaphoreType.DMA((3,)),
        ],
        compiler_params=pltpu.CompilerParams(
            collective_id=0, vmem_limit_bytes=_VMEM_LIMIT),
    )(x, Wdkv, Wuk, Wuv, Wq, Wqr, Wkr, Wo)
